# Initial kernel scaffold; baseline (speedup 1.0000x reference)
import jax
import jax.numpy as jnp
from jax.experimental import pallas as pl


def kernel(sent, mask, target, word_table, mask_table):
    raise NotImplementedError("write your pallas kernel here")


# speedup vs baseline: 2.0373x; 2.0373x over previous
import functools

import jax
import jax.numpy as jnp
from jax import lax
from jax.experimental import pallas as pl
from jax.experimental.pallas import tpu as pltpu, tpu_sc as plsc

VOCAB = 1000000
EMB = 64
MD = 50
B = 4096
L = 200
BL = B * L
OUT_D = EMB + MD

NC = 2
NS = 16
NW = NC * NS

SUB = 128
JJ = 4
CHUNK = SUB * JJ
SG = 8
SG_PER_CHUNK = CHUNK // SG
ROWS_PER_W = BL // NW
N_CHUNKS = ROWS_PER_W // CHUNK
TGT_PER_W = B // NW

_mesh = plsc.VectorSubcoreMesh(
    core_axis_name="c", subcore_axis_name="s", num_cores=NC, num_subcores=NS
)


@functools.partial(
    pl.kernel,
    out_type=(
        jax.ShapeDtypeStruct((BL, OUT_D), jnp.float32),
        jax.ShapeDtypeStruct((B, EMB), jnp.float32),
    ),
    mesh=_mesh,
    compiler_params=pltpu.CompilerParams(use_tc_tiling_on_sc=False),
    scratch_types=[
        pltpu.VMEM((JJ, SUB), jnp.int32),
        pltpu.VMEM((SG_PER_CHUNK,), jnp.int32),
        pltpu.VMEM((JJ, SUB, EMB), jnp.float32),
        pltpu.VMEM((SG_PER_CHUNK, SG, MD), jnp.float32),
        pltpu.VMEM((1, TGT_PER_W), jnp.int32),
        pltpu.VMEM((TGT_PER_W, EMB), jnp.float32),
        pltpu.SemaphoreType.DMA,
        pltpu.SemaphoreType.DMA,
    ],
)
def _simple_cat_sc(
    sent_ref, codes_ref, target_ref, wt_ref, pat_ref,
    out_ref, tgt_ref,
    idx_v, codes_v, word_v, mask_v, tidx_v, trow_v, sem, wsem,
):
    c = lax.axis_index("c")
    s = lax.axis_index("s")
    wid = s * NC + c
    crow0 = wid * (ROWS_PER_W // SUB)
    sg0 = wid * (ROWS_PER_W // SG)

    def chunk_body(i, carry):
        crow = crow0 + i * JJ
        sg_base = sg0 + i * SG_PER_CHUNK
        row_base = sg_base * SG
        pltpu.sync_copy(sent_ref.at[pl.ds(crow, JJ)], idx_v)
        pltpu.sync_copy(codes_ref.at[pl.ds(sg_base, SG_PER_CHUNK)], codes_v)
        copies = [pltpu.async_copy(pat_ref.at[codes_v], mask_v, sem)]
        for j in range(JJ):
            copies.append(pltpu.async_copy(wt_ref.at[idx_v.at[j]], word_v.at[j], sem))
        for cp in copies:
            cp.wait()
        for j in range(JJ):
            pltpu.sync_copy(
                word_v.at[j],
                out_ref.at[pl.ds(row_base + j * SUB, SUB), pl.ds(0, EMB)],
            )

        def mask_write(g, carry2):
            pltpu.async_copy(
                mask_v.at[g],
                out_ref.at[pl.ds(row_base + g * SG, SG), pl.ds(EMB, MD)],
                wsem,
            )
            return carry2

        lax.fori_loop(0, SG_PER_CHUNK, mask_write, 0)

        def mask_drain(g, carry2):
            pltpu.make_async_copy(
                mask_v.at[0],
                out_ref.at[pl.ds(row_base, SG), pl.ds(EMB, MD)],
                wsem,
            ).wait()
            return carry2

        lax.fori_loop(0, SG_PER_CHUNK, mask_drain, 0)
        return carry

    lax.fori_loop(0, N_CHUNKS, chunk_body, 0)

    pltpu.sync_copy(target_ref.at[pl.ds(wid, 1)], tidx_v)
    pltpu.async_copy(wt_ref.at[tidx_v.at[0]], trow_v, sem).wait()
    pltpu.sync_copy(trow_v, tgt_ref.at[pl.ds(wid * TGT_PER_W, TGT_PER_W)])


def kernel(sent, mask, target, word_table, mask_table):
    sent2 = sent.reshape(BL // SUB, SUB)
    tgt2 = target.reshape(NW, TGT_PER_W)
    bits = mask.astype(jnp.int32).reshape(BL // SG, SG)
    codes = jnp.sum(bits << jnp.arange(SG, dtype=jnp.int32)[None, :], axis=1)
    pat_bits = (jnp.arange(256, dtype=jnp.int32)[:, None] >> jnp.arange(SG, dtype=jnp.int32)[None, :]) & 1
    patterns = jnp.take(mask_table, pat_bits, axis=0)
    out_flat, tgt_out = _simple_cat_sc(sent2, codes, tgt2, word_table, patterns)
    return out_flat.reshape(B, L, OUT_D), tgt_out


# speedup vs baseline: 2.1522x; 1.0564x over previous
import functools

import jax
import jax.numpy as jnp
from jax import lax
from jax.experimental import pallas as pl
from jax.experimental.pallas import tpu as pltpu, tpu_sc as plsc

VOCAB = 1000000
EMB = 64
MD = 50
B = 4096
L = 200
BL = B * L
OUT_D = EMB + MD

NC = 2
NS = 16
NW = NC * NS

LA = 96
LB = L - LA
SG = 8
SG_PER_S = L // SG
SENT_PER_W = B // NW
TGT_PER_W = B // NW

_mesh = plsc.VectorSubcoreMesh(
    core_axis_name="c", subcore_axis_name="s", num_cores=NC, num_subcores=NS
)


@functools.partial(
    pl.kernel,
    out_type=(
        jax.ShapeDtypeStruct((B, L, OUT_D), jnp.float32),
        jax.ShapeDtypeStruct((B, EMB), jnp.float32),
    ),
    mesh=_mesh,
    compiler_params=pltpu.CompilerParams(use_tc_tiling_on_sc=False),
    scratch_types=[
        pltpu.VMEM((SENT_PER_W, L), jnp.int32),
        pltpu.VMEM((SENT_PER_W, SG_PER_S), jnp.int32),
        pltpu.VMEM((LA, EMB), jnp.float32),
        pltpu.VMEM((LB, EMB), jnp.float32),
        pltpu.VMEM((LA, EMB), jnp.float32),
        pltpu.VMEM((LB, EMB), jnp.float32),
        pltpu.VMEM((SG_PER_S, SG, MD), jnp.float32),
        pltpu.VMEM((SG_PER_S, SG, MD), jnp.float32),
        pltpu.VMEM((TGT_PER_W,), jnp.int32),
        pltpu.VMEM((TGT_PER_W, EMB), jnp.float32),
        pltpu.SemaphoreType.DMA,
        pltpu.SemaphoreType.DMA,
        pltpu.SemaphoreType.DMA,
        pltpu.SemaphoreType.DMA,
    ],
)
def _simple_cat_sc(
    sent_ref, codes_ref, target_ref, wt_ref, pat_ref,
    out_ref, tgt_ref,
    idx_all, codes_all, wa0, wb0, wa1, wb1, mv0, mv1,
    tidx_v, trow_v, gsem0, gsem1, wsem0, wsem1,
):
    c = lax.axis_index("c")
    s = lax.axis_index("s")
    wid = s * NC + c
    b0 = wid * SENT_PER_W

    wa_b = (wa0, wa1)
    wb_b = (wb0, wb1)
    mv_b = (mv0, mv1)
    gsem_b = (gsem0, gsem1)
    wsem_b = (wsem0, wsem1)

    pltpu.sync_copy(sent_ref.at[pl.ds(b0, SENT_PER_W)], idx_all)
    pltpu.sync_copy(codes_ref.at[pl.ds(b0, SENT_PER_W)], codes_all)

    def fire_gathers(si, k):
        pltpu.async_copy(wt_ref.at[idx_all.at[si, pl.ds(0, LA)]], wa_b[k], gsem_b[k])
        pltpu.async_copy(wt_ref.at[idx_all.at[si, pl.ds(LA, LB)]], wb_b[k], gsem_b[k])
        pltpu.async_copy(pat_ref.at[codes_all.at[si]], mv_b[k], gsem_b[k])

    def drain_gathers(k):
        pltpu.make_async_copy(wt_ref.at[idx_all.at[0, pl.ds(0, LA)]], wa_b[k], gsem_b[k]).wait()
        pltpu.make_async_copy(wt_ref.at[idx_all.at[0, pl.ds(LA, LB)]], wb_b[k], gsem_b[k]).wait()
        pltpu.make_async_copy(pat_ref.at[codes_all.at[0]], mv_b[k], gsem_b[k]).wait()

    def fire_writes(si, k):
        bb = b0 + si
        pltpu.async_copy(wa_b[k], out_ref.at[bb, pl.ds(0, LA), pl.ds(0, EMB)], wsem_b[k])
        pltpu.async_copy(wb_b[k], out_ref.at[bb, pl.ds(LA, LB), pl.ds(0, EMB)], wsem_b[k])

        def mask_write(g, carry):
            pltpu.async_copy(
                mv_b[k].at[g],
                out_ref.at[bb, pl.ds(g * SG, SG), pl.ds(EMB, MD)],
                wsem_b[k],
            )
            return carry

        lax.fori_loop(0, SG_PER_S, mask_write, 0)

    def drain_writes(k):
        pltpu.make_async_copy(wa_b[k], out_ref.at[b0, pl.ds(0, LA), pl.ds(0, EMB)], wsem_b[k]).wait()
        pltpu.make_async_copy(wb_b[k], out_ref.at[b0, pl.ds(LA, LB), pl.ds(0, EMB)], wsem_b[k]).wait()
        pltpu.make_async_copy(pat_ref.at[pl.ds(0, SG_PER_S)], mv_b[k], wsem_b[k]).wait()

    fire_gathers(0, 0)
    fire_gathers(1, 1)

    def pair_body(io, carry):
        s0 = 2 * io
        drain_gathers(0)
        fire_writes(s0, 0)
        drain_gathers(1)
        fire_writes(s0 + 1, 1)
        drain_writes(0)

        @pl.when(s0 + 2 < SENT_PER_W)
        def _():
            fire_gathers(s0 + 2, 0)

        drain_writes(1)

        @pl.when(s0 + 3 < SENT_PER_W)
        def _():
            fire_gathers(s0 + 3, 1)

        return carry

    lax.fori_loop(0, SENT_PER_W // 2, pair_body, 0)

    pltpu.sync_copy(target_ref.at[pl.ds(wid * TGT_PER_W, TGT_PER_W)], tidx_v)
    pltpu.async_copy(wt_ref.at[tidx_v], trow_v, gsem0).wait()
    pltpu.sync_copy(trow_v, tgt_ref.at[pl.ds(wid * TGT_PER_W, TGT_PER_W)])


def kernel(sent, mask, target, word_table, mask_table):
    bits = mask.astype(jnp.int32).reshape(B, SG_PER_S, SG)
    codes = jnp.sum(bits << jnp.arange(SG, dtype=jnp.int32)[None, None, :], axis=2)
    pat_bits = (jnp.arange(256, dtype=jnp.int32)[:, None] >> jnp.arange(SG, dtype=jnp.int32)[None, :]) & 1
    patterns = jnp.take(mask_table, pat_bits, axis=0)
    return _simple_cat_sc(sent, codes, target, word_table, patterns)


# speedup vs baseline: 3.0804x; 1.4313x over previous
import functools

import jax
import jax.numpy as jnp
from jax import lax
from jax.experimental import pallas as pl
from jax.experimental.pallas import tpu as pltpu, tpu_sc as plsc

VOCAB = 1000000
EMB = 64
MD = 50
B = 4096
L = 200
BL = B * L
OUT_D = EMB + MD

NC = 2
NS = 16
NW = NC * NS

LA = 96
LB = L - LA
SG = 8
MDP = 56
SG_PER_S = L // SG
SENT_PER_W = B // NW
TGT_PER_W = B // NW

_mesh = plsc.VectorSubcoreMesh(
    core_axis_name="c", subcore_axis_name="s", num_cores=NC, num_subcores=NS
)


@functools.partial(
    pl.kernel,
    out_type=(
        jax.ShapeDtypeStruct((BL, 128), jnp.float32),
        jax.ShapeDtypeStruct((B, EMB), jnp.float32),
    ),
    mesh=_mesh,
    compiler_params=pltpu.CompilerParams(use_tc_tiling_on_sc=False),
    scratch_types=[
        pltpu.VMEM((SENT_PER_W, L), jnp.int32),
        pltpu.VMEM((SENT_PER_W, SG_PER_S), jnp.int32),
        pltpu.VMEM((LA, EMB), jnp.float32),
        pltpu.VMEM((LB, EMB), jnp.float32),
        pltpu.VMEM((LA, EMB), jnp.float32),
        pltpu.VMEM((LB, EMB), jnp.float32),
        pltpu.VMEM((SG_PER_S, SG, MDP), jnp.float32),
        pltpu.VMEM((SG_PER_S, SG, MDP), jnp.float32),
        pltpu.VMEM((TGT_PER_W,), jnp.int32),
        pltpu.VMEM((TGT_PER_W, EMB), jnp.float32),
        pltpu.SemaphoreType.DMA,
        pltpu.SemaphoreType.DMA,
        pltpu.SemaphoreType.DMA,
        pltpu.SemaphoreType.DMA,
    ],
)
def _simple_cat_sc(
    sent_ref, codes_ref, target_ref, wt_ref, pat_ref,
    out_ref, tgt_ref,
    idx_all, codes_all, wa0, wb0, wa1, wb1, mv0, mv1,
    tidx_v, trow_v, gsem0, gsem1, wsem0, wsem1,
):
    c = lax.axis_index("c")
    s = lax.axis_index("s")
    wid = s * NC + c
    b0 = wid * SENT_PER_W

    wa_b = (wa0, wa1)
    wb_b = (wb0, wb1)
    mv_b = (mv0, mv1)
    gsem_b = (gsem0, gsem1)
    wsem_b = (wsem0, wsem1)

    pltpu.sync_copy(sent_ref.at[pl.ds(b0, SENT_PER_W)], idx_all)
    pltpu.sync_copy(codes_ref.at[pl.ds(b0, SENT_PER_W)], codes_all)

    def fire_gathers(si, k):
        pltpu.async_copy(wt_ref.at[idx_all.at[si, pl.ds(0, LA)]], wa_b[k], gsem_b[k])
        pltpu.async_copy(wt_ref.at[idx_all.at[si, pl.ds(LA, LB)]], wb_b[k], gsem_b[k])
        pltpu.async_copy(pat_ref.at[codes_all.at[si]], mv_b[k], gsem_b[k])

    def drain_gathers(k):
        pltpu.make_async_copy(wt_ref.at[idx_all.at[0, pl.ds(0, LA)]], wa_b[k], gsem_b[k]).wait()
        pltpu.make_async_copy(wt_ref.at[idx_all.at[0, pl.ds(LA, LB)]], wb_b[k], gsem_b[k]).wait()
        pltpu.make_async_copy(pat_ref.at[codes_all.at[0]], mv_b[k], gsem_b[k]).wait()

    def fire_writes(si, k):
        r0 = (b0 + si) * L
        pltpu.async_copy(wa_b[k], out_ref.at[pl.ds(r0, LA), pl.ds(0, EMB)], wsem_b[k])
        pltpu.async_copy(wb_b[k], out_ref.at[pl.ds(r0 + LA, LB), pl.ds(0, EMB)], wsem_b[k])

        def mask_write(g, carry):
            pltpu.async_copy(
                mv_b[k].at[g],
                out_ref.at[pl.ds(r0 + g * SG, SG), pl.ds(EMB, MDP)],
                wsem_b[k],
            )
            return carry

        lax.fori_loop(0, SG_PER_S, mask_write, 0)

    def drain_writes(k):
        pltpu.make_async_copy(wa_b[k], out_ref.at[pl.ds(b0 * L, LA), pl.ds(0, EMB)], wsem_b[k]).wait()
        pltpu.make_async_copy(wb_b[k], out_ref.at[pl.ds(b0 * L, LB), pl.ds(0, EMB)], wsem_b[k]).wait()
        pltpu.make_async_copy(pat_ref.at[pl.ds(0, SG_PER_S)], mv_b[k], wsem_b[k]).wait()

    fire_gathers(0, 0)
    fire_gathers(1, 1)

    def pair_body(io, carry):
        s0 = 2 * io
        drain_gathers(0)
        fire_writes(s0, 0)
        drain_gathers(1)
        fire_writes(s0 + 1, 1)
        drain_writes(0)

        @pl.when(s0 + 2 < SENT_PER_W)
        def _():
            fire_gathers(s0 + 2, 0)

        drain_writes(1)

        @pl.when(s0 + 3 < SENT_PER_W)
        def _():
            fire_gathers(s0 + 3, 1)

        return carry

    lax.fori_loop(0, SENT_PER_W // 2, pair_body, 0)

    pltpu.sync_copy(target_ref.at[pl.ds(wid * TGT_PER_W, TGT_PER_W)], tidx_v)
    pltpu.async_copy(wt_ref.at[tidx_v], trow_v, gsem0).wait()
    pltpu.sync_copy(trow_v, tgt_ref.at[pl.ds(wid * TGT_PER_W, TGT_PER_W)])


def kernel(sent, mask, target, word_table, mask_table):
    bits = mask.astype(jnp.int32).reshape(B, SG_PER_S, SG)
    codes = jnp.sum(bits << jnp.arange(SG, dtype=jnp.int32)[None, None, :], axis=2)
    pat_bits = (jnp.arange(256, dtype=jnp.int32)[:, None] >> jnp.arange(SG, dtype=jnp.int32)[None, :]) & 1
    patterns = jnp.take(mask_table, pat_bits, axis=0)
    patterns = jnp.pad(patterns, ((0, 0), (0, 0), (0, MDP - MD)))
    out128, tgt_out = _simple_cat_sc(sent, codes, target, word_table, patterns)
    return out128[:, :OUT_D].reshape(B, L, OUT_D), tgt_out


# speedup vs baseline: 3.0929x; 1.0041x over previous
import functools

import jax
import jax.numpy as jnp
from jax import lax
from jax.experimental import pallas as pl
from jax.experimental.pallas import tpu as pltpu, tpu_sc as plsc

VOCAB = 1000000
EMB = 64
MD = 50
B = 4096
L = 200
BL = B * L
OUT_D = EMB + MD

NC = 2
NS = 16
NW = NC * NS

LA = 96
LB = L - LA
SG = 8
MDP = 56
SG_PER_S = L // SG
SENT_PER_W = B // NW
TGT_PER_W = B // NW

_mesh = plsc.VectorSubcoreMesh(
    core_axis_name="c", subcore_axis_name="s", num_cores=NC, num_subcores=NS
)


@functools.partial(
    pl.kernel,
    out_type=(
        jax.ShapeDtypeStruct((BL, 128), jnp.float32),
        jax.ShapeDtypeStruct((B, EMB), jnp.float32),
    ),
    mesh=_mesh,
    compiler_params=pltpu.CompilerParams(use_tc_tiling_on_sc=False),
    scratch_types=[
        pltpu.VMEM((SENT_PER_W, L), jnp.int32),
        pltpu.VMEM((SENT_PER_W, SG_PER_S), jnp.int32),
        pltpu.VMEM((LA, EMB), jnp.float32),
        pltpu.VMEM((LB, EMB), jnp.float32),
        pltpu.VMEM((LA, EMB), jnp.float32),
        pltpu.VMEM((LB, EMB), jnp.float32),
        pltpu.VMEM((LA, EMB), jnp.float32),
        pltpu.VMEM((LB, EMB), jnp.float32),
        pltpu.VMEM((SG_PER_S, SG, MDP), jnp.float32),
        pltpu.VMEM((SG_PER_S, SG, MDP), jnp.float32),
        pltpu.VMEM((SG_PER_S, SG, MDP), jnp.float32),
        pltpu.VMEM((TGT_PER_W,), jnp.int32),
        pltpu.VMEM((TGT_PER_W, EMB), jnp.float32),
        pltpu.SemaphoreType.DMA,
        pltpu.SemaphoreType.DMA,
        pltpu.SemaphoreType.DMA,
        pltpu.SemaphoreType.DMA,
        pltpu.SemaphoreType.DMA,
        pltpu.SemaphoreType.DMA,
    ],
)
def _simple_cat_sc(
    sent_ref, codes_ref, target_ref, wt_ref, pat_ref,
    out_ref, tgt_ref,
    idx_all, codes_all, wa0, wb0, wa1, wb1, wa2, wb2, mv0, mv1, mv2,
    tidx_v, trow_v, gsem0, gsem1, gsem2, wsem0, wsem1, wsem2,
):
    c = lax.axis_index("c")
    s = lax.axis_index("s")
    wid = s * NC + c
    b0 = wid * SENT_PER_W

    wa_b = (wa0, wa1, wa2)
    wb_b = (wb0, wb1, wb2)
    mv_b = (mv0, mv1, mv2)
    gsem_b = (gsem0, gsem1, gsem2)
    wsem_b = (wsem0, wsem1, wsem2)

    pltpu.sync_copy(sent_ref.at[pl.ds(b0, SENT_PER_W)], idx_all)
    pltpu.sync_copy(codes_ref.at[pl.ds(b0, SENT_PER_W)], codes_all)

    def fire_gathers(si, k):
        pltpu.async_copy(wt_ref.at[idx_all.at[si, pl.ds(0, LA)]], wa_b[k], gsem_b[k])
        pltpu.async_copy(wt_ref.at[idx_all.at[si, pl.ds(LA, LB)]], wb_b[k], gsem_b[k])
        pltpu.async_copy(pat_ref.at[codes_all.at[si]], mv_b[k], gsem_b[k])

    def drain_gathers(k):
        pltpu.make_async_copy(wt_ref.at[idx_all.at[0, pl.ds(0, LA)]], wa_b[k], gsem_b[k]).wait()
        pltpu.make_async_copy(wt_ref.at[idx_all.at[0, pl.ds(LA, LB)]], wb_b[k], gsem_b[k]).wait()
        pltpu.make_async_copy(pat_ref.at[codes_all.at[0]], mv_b[k], gsem_b[k]).wait()

    def fire_writes(si, k):
        r0 = (b0 + si) * L
        pltpu.async_copy(wa_b[k], out_ref.at[pl.ds(r0, LA), pl.ds(0, EMB)], wsem_b[k])
        pltpu.async_copy(wb_b[k], out_ref.at[pl.ds(r0 + LA, LB), pl.ds(0, EMB)], wsem_b[k])

        def mask_write(g, carry):
            pltpu.async_copy(
                mv_b[k].at[g],
                out_ref.at[pl.ds(r0 + g * SG, SG), pl.ds(EMB, MDP)],
                wsem_b[k],
            )
            return carry

        lax.fori_loop(0, SG_PER_S, mask_write, 0)

    def drain_writes(k):
        pltpu.make_async_copy(wa_b[k], out_ref.at[pl.ds(b0 * L, LA), pl.ds(0, EMB)], wsem_b[k]).wait()
        pltpu.make_async_copy(wb_b[k], out_ref.at[pl.ds(b0 * L, LB), pl.ds(0, EMB)], wsem_b[k]).wait()
        pltpu.make_async_copy(pat_ref.at[pl.ds(0, SG_PER_S)], mv_b[k], wsem_b[k]).wait()

    fire_gathers(0, 0)
    fire_gathers(1, 1)
    fire_gathers(2, 2)

    def tri_body(io, carry):
        s0 = 3 * io
        for j in range(3):
            si = s0 + j

            @pl.when(si < SENT_PER_W)
            def _(si=si, j=j):
                drain_gathers(j)
                fire_writes(si, j)

        for j in range(3):
            si = s0 + j

            @pl.when(si < SENT_PER_W)
            def _(si=si, j=j):
                drain_writes(j)

                @pl.when(si + 3 < SENT_PER_W)
                def _():
                    fire_gathers(si + 3, j)

        return carry

    lax.fori_loop(0, (SENT_PER_W + 2) // 3, tri_body, 0)

    pltpu.sync_copy(target_ref.at[pl.ds(wid * TGT_PER_W, TGT_PER_W)], tidx_v)
    pltpu.async_copy(wt_ref.at[tidx_v], trow_v, gsem0).wait()
    pltpu.sync_copy(trow_v, tgt_ref.at[pl.ds(wid * TGT_PER_W, TGT_PER_W)])


def kernel(sent, mask, target, word_table, mask_table):
    bits = mask.astype(jnp.int32).reshape(B, SG_PER_S, SG)
    codes = jnp.sum(bits << jnp.arange(SG, dtype=jnp.int32)[None, None, :], axis=2)
    pat_bits = (jnp.arange(256, dtype=jnp.int32)[:, None] >> jnp.arange(SG, dtype=jnp.int32)[None, :]) & 1
    patterns = jnp.take(mask_table, pat_bits, axis=0)
    patterns = jnp.pad(patterns, ((0, 0), (0, 0), (0, MDP - MD)))
    out128, tgt_out = _simple_cat_sc(sent, codes, target, word_table, patterns)
    return out128[:, :OUT_D].reshape(B, L, OUT_D), tgt_out
